# baseline (speedup 1.0000x reference)
import functools

import jax
import jax.numpy as jnp
from jax import lax
from jax.experimental import pallas as pl
from jax.experimental.pallas import tpu as pltpu
from jax.experimental.pallas import tpu_sc as plsc

N_NODES = 10000
N_EDGES = 320000
D = 128

NC = 2
NS = 16
NW = NC * NS

CB = 128
NCHUNK = 80
EPW = CB * NCHUNK
E_PAD = EPW * NW
DUMP_ROW = N_NODES
NPAD = 10240
RPT = NPAD // NS

_MESH = dict(core_axis_name="c", subcore_axis_name="s")


def _sc_agg_body(feats, src3d, dst3d, z128, agg_out,
                 srcv, dstv, rows, aggs, gsem):
    cid = lax.axis_index("c")
    sid = lax.axis_index("s")
    wid = sid * NC + cid

    r0 = sid * RPT
    pltpu.sync_copy(z128.at[pl.ds(r0, RPT)], aggs.at[pl.ds(r0, RPT)])
    plsc.subcore_barrier()

    pltpu.sync_copy(src3d.at[wid, 0], srcv.at[0])
    pltpu.sync_copy(dst3d.at[wid, 0], dstv.at[0])
    pltpu.async_copy(feats.at[srcv.at[0]], rows.at[0], gsem)

    def chunk(j, carry):
        p = lax.rem(j, 2)
        q = lax.rem(j + 1, 2)
        jn = lax.min(j + 1, NCHUNK - 1)
        pltpu.sync_copy(src3d.at[wid, jn], srcv.at[q])
        pltpu.sync_copy(dst3d.at[wid, jn], dstv.at[q])
        pltpu.make_async_copy(feats.at[srcv.at[p]], rows.at[p], gsem).wait()
        pltpu.async_copy(feats.at[srcv.at[q]], rows.at[q], gsem)
        pltpu.sync_copy(rows.at[p], aggs.at[dstv.at[p]], add=True)
        return carry

    lax.fori_loop(0, NCHUNK, chunk, 0)
    pltpu.make_async_copy(feats.at[srcv.at[0]], rows.at[0], gsem).wait()
    plsc.subcore_barrier()

    pltpu.sync_copy(aggs.at[pl.ds(r0, RPT)],
                    agg_out.at[cid, pl.ds(r0, RPT)])


_sc_agg = pl.kernel(
    _sc_agg_body,
    out_type=[jax.ShapeDtypeStruct((NC, NPAD, D), jnp.float32)],
    mesh=plsc.VectorSubcoreMesh(**_MESH),
    scratch_types=[
        pltpu.VMEM((2, CB), jnp.int32),
        pltpu.VMEM((2, CB), jnp.int32),
        pltpu.VMEM((2, CB, D), jnp.float32),
        pltpu.VMEM_SHARED((NPAD, D), jnp.float32),
        pltpu.SemaphoreType.DMA,
    ],
)


def _sc_deg_body(dst3d, z128, ones_h, deg_out, dstv, ones, degs):
    cid = lax.axis_index("c")
    sid = lax.axis_index("s")
    wid = sid * NC + cid

    r0 = sid * RPT
    pltpu.sync_copy(z128.at[pl.ds(r0, RPT)], degs.at[pl.ds(r0, RPT)])
    pltpu.sync_copy(ones_h, ones)
    plsc.subcore_barrier()

    def chunk(j, carry):
        pltpu.sync_copy(dst3d.at[wid, j], dstv)
        pltpu.sync_copy(ones, degs.at[dstv], add=True)
        return carry

    lax.fori_loop(0, NCHUNK, chunk, 0)
    plsc.subcore_barrier()

    pltpu.sync_copy(degs.at[pl.ds(r0, RPT)],
                    deg_out.at[cid, pl.ds(r0, RPT)])


_sc_deg = pl.kernel(
    _sc_deg_body,
    out_type=[jax.ShapeDtypeStruct((NC, NPAD, D), jnp.float32)],
    mesh=plsc.VectorSubcoreMesh(**_MESH),
    scratch_types=[
        pltpu.VMEM((CB,), jnp.int32),
        pltpu.VMEM((CB, D), jnp.float32),
        pltpu.VMEM_SHARED((NPAD, D), jnp.float32),
    ],
)


def _tc_body(relu, xb, ap, dp, wl, wr, bb, ob):
    agg = ap[0] + ap[1]
    deg = jnp.maximum(dp[0, :, :1] + dp[1, :, :1], 1.0)
    m = agg / deg
    y = (jax.lax.dot(m, wl[...], preferred_element_type=jnp.float32)
         + jax.lax.dot(xb[...], wr[...], preferred_element_type=jnp.float32)
         + bb[...])
    ob[...] = jnp.maximum(y, 0.0) if relu else y


def _tc_layer(x, agg_p, deg_p, Wl, Wr, b, relu):
    blk = 1000
    grid = (N_NODES // blk,)
    return pl.pallas_call(
        functools.partial(_tc_body, relu),
        grid=grid,
        in_specs=[
            pl.BlockSpec((blk, D), lambda i: (i, 0)),
            pl.BlockSpec((2, blk, D), lambda i: (0, i, 0)),
            pl.BlockSpec((2, blk, D), lambda i: (0, i, 0)),
            pl.BlockSpec((D, D), lambda i: (0, 0)),
            pl.BlockSpec((D, D), lambda i: (0, 0)),
            pl.BlockSpec((1, D), lambda i: (0, 0)),
        ],
        out_specs=pl.BlockSpec((blk, D), lambda i: (i, 0)),
        out_shape=jax.ShapeDtypeStruct((N_NODES, D), jnp.float32),
    )(x, agg_p, deg_p, Wl, Wr, b.reshape(1, D))


def kernel(x, edge_index, Wl1, Wr1, b1, Wl2, Wr2, b2):
    edge_index = edge_index.astype(jnp.int32)
    src = edge_index[0]
    dst = edge_index[1]
    pad = E_PAD - N_EDGES
    srcp = jnp.concatenate([src, jnp.zeros((pad,), jnp.int32)])
    dump = DUMP_ROW + jnp.arange(pad, dtype=jnp.int32) % (NPAD - N_NODES)
    dstp = jnp.concatenate([dst, dump])
    src3d = srcp.reshape(NW, NCHUNK, CB)
    dst3d = dstp.reshape(NW, NCHUNK, CB)
    z128 = jnp.zeros((NPAD, D), jnp.float32)
    ones_h = jnp.ones((CB, D), jnp.float32)

    (deg,) = _sc_deg(dst3d, z128, ones_h)
    (agg1,) = _sc_agg(x, src3d, dst3d, z128)
    h = _tc_layer(x, agg1, deg, Wl1, Wr1, b1, relu=True)
    (agg2,) = _sc_agg(h, src3d, dst3d, z128)
    out = _tc_layer(h, agg2, deg, Wl2, Wr2, b2, relu=False)
    return out


# speedup vs baseline: 1.0340x; 1.0340x over previous
import functools

import jax
import jax.numpy as jnp
from jax import lax
from jax.experimental import pallas as pl
from jax.experimental.pallas import tpu as pltpu
from jax.experimental.pallas import tpu_sc as plsc

N_NODES = 10000
N_EDGES = 320000
D = 128

NC = 2
NS = 16
NW = NC * NS

CB = 128
NCHUNK = 80
EPW = CB * NCHUNK
E_PAD = EPW * NW
DUMP_ROW = N_NODES
NPAD = 10240
RPT = NPAD // NS

_MESH = dict(core_axis_name="c", subcore_axis_name="s")


def _sc_agg_body(feats, src3d, dst3d, z128, agg_out,
                 srcv, dstv, rows, aggs, gsem, asem):
    cid = lax.axis_index("c")
    sid = lax.axis_index("s")
    wid = sid * NC + cid

    r0 = sid * RPT
    pltpu.sync_copy(z128.at[pl.ds(r0, RPT)], aggs.at[pl.ds(r0, RPT)])
    plsc.subcore_barrier()

    pltpu.sync_copy(src3d.at[wid, 0], srcv.at[0])
    pltpu.sync_copy(dst3d.at[wid, 0], dstv.at[0])
    pltpu.async_copy(feats.at[srcv.at[0]], rows.at[0], gsem)

    def chunk(j, carry):
        p = lax.rem(j, 2)
        q = lax.rem(j + 1, 2)
        jn = lax.min(j + 1, NCHUNK - 1)
        pltpu.sync_copy(src3d.at[wid, jn], srcv.at[q])
        pltpu.sync_copy(dst3d.at[wid, jn], dstv.at[lax.rem(j + 1, 4)])
        pltpu.make_async_copy(feats.at[srcv.at[p]], rows.at[p], gsem).wait()

        @pl.when(j >= 1)
        def _():
            pltpu.make_async_copy(
                rows.at[q], aggs.at[dstv.at[lax.rem(j + 3, 4)]], asem).wait()

        pltpu.async_copy(feats.at[srcv.at[q]], rows.at[q], gsem)
        pltpu.async_copy(rows.at[p], aggs.at[dstv.at[lax.rem(j, 4)]], asem,
                         add=True)
        return carry

    lax.fori_loop(0, NCHUNK, chunk, 0)
    pltpu.make_async_copy(
        rows.at[lax.rem(NCHUNK - 1, 2)],
        aggs.at[dstv.at[lax.rem(NCHUNK - 1, 4)]], asem).wait()
    pltpu.make_async_copy(feats.at[srcv.at[0]], rows.at[0], gsem).wait()
    plsc.subcore_barrier()

    pltpu.sync_copy(aggs.at[pl.ds(r0, RPT)],
                    agg_out.at[cid, pl.ds(r0, RPT)])


_sc_agg = pl.kernel(
    _sc_agg_body,
    out_type=[jax.ShapeDtypeStruct((NC, NPAD, D), jnp.float32)],
    mesh=plsc.VectorSubcoreMesh(**_MESH),
    scratch_types=[
        pltpu.VMEM((2, CB), jnp.int32),
        pltpu.VMEM((4, CB), jnp.int32),
        pltpu.VMEM((2, CB, D), jnp.float32),
        pltpu.VMEM_SHARED((NPAD, D), jnp.float32),
        pltpu.SemaphoreType.DMA,
        pltpu.SemaphoreType.DMA,
    ],
)


def _sc_deg_body(dst3d, z128, ones_h, deg_out, dstv, ones, degs, asem):
    cid = lax.axis_index("c")
    sid = lax.axis_index("s")
    wid = sid * NC + cid

    r0 = sid * RPT
    pltpu.sync_copy(z128.at[pl.ds(r0, RPT)], degs.at[pl.ds(r0, RPT)])
    pltpu.sync_copy(ones_h, ones)
    plsc.subcore_barrier()

    pltpu.sync_copy(dst3d.at[wid, 0], dstv.at[0])

    def chunk(j, carry):
        jn = lax.min(j + 1, NCHUNK - 1)
        pltpu.sync_copy(dst3d.at[wid, jn], dstv.at[lax.rem(j + 1, 4)])

        @pl.when(j >= 1)
        def _():
            pltpu.make_async_copy(
                ones, degs.at[dstv.at[lax.rem(j + 3, 4)]], asem).wait()

        pltpu.async_copy(ones, degs.at[dstv.at[lax.rem(j, 4)]], asem,
                         add=True)
        return carry

    lax.fori_loop(0, NCHUNK, chunk, 0)
    pltpu.make_async_copy(
        ones, degs.at[dstv.at[lax.rem(NCHUNK - 1, 4)]], asem).wait()
    plsc.subcore_barrier()

    pltpu.sync_copy(degs.at[pl.ds(r0, RPT)],
                    deg_out.at[cid, pl.ds(r0, RPT)])


_sc_deg = pl.kernel(
    _sc_deg_body,
    out_type=[jax.ShapeDtypeStruct((NC, NPAD, D), jnp.float32)],
    mesh=plsc.VectorSubcoreMesh(**_MESH),
    scratch_types=[
        pltpu.VMEM((4, CB), jnp.int32),
        pltpu.VMEM((CB, D), jnp.float32),
        pltpu.VMEM_SHARED((NPAD, D), jnp.float32),
        pltpu.SemaphoreType.DMA,
    ],
)


def _tc_body(relu, xb, ap, dp, wl, wr, bb, ob):
    agg = ap[0] + ap[1]
    deg = jnp.maximum(dp[0, :, :1] + dp[1, :, :1], 1.0)
    m = agg / deg
    y = (jax.lax.dot(m, wl[...], preferred_element_type=jnp.float32)
         + jax.lax.dot(xb[...], wr[...], preferred_element_type=jnp.float32)
         + bb[...])
    ob[...] = jnp.maximum(y, 0.0) if relu else y


def _tc_layer(x, agg_p, deg_p, Wl, Wr, b, relu):
    blk = 1000
    grid = (N_NODES // blk,)
    return pl.pallas_call(
        functools.partial(_tc_body, relu),
        grid=grid,
        in_specs=[
            pl.BlockSpec((blk, D), lambda i: (i, 0)),
            pl.BlockSpec((2, blk, D), lambda i: (0, i, 0)),
            pl.BlockSpec((2, blk, D), lambda i: (0, i, 0)),
            pl.BlockSpec((D, D), lambda i: (0, 0)),
            pl.BlockSpec((D, D), lambda i: (0, 0)),
            pl.BlockSpec((1, D), lambda i: (0, 0)),
        ],
        out_specs=pl.BlockSpec((blk, D), lambda i: (i, 0)),
        out_shape=jax.ShapeDtypeStruct((N_NODES, D), jnp.float32),
    )(x, agg_p, deg_p, Wl, Wr, b.reshape(1, D))


def kernel(x, edge_index, Wl1, Wr1, b1, Wl2, Wr2, b2):
    edge_index = edge_index.astype(jnp.int32)
    src = edge_index[0]
    dst = edge_index[1]
    pad = E_PAD - N_EDGES
    srcp = jnp.concatenate([src, jnp.zeros((pad,), jnp.int32)])
    dump = DUMP_ROW + jnp.arange(pad, dtype=jnp.int32) % (NPAD - N_NODES)
    dstp = jnp.concatenate([dst, dump])
    src3d = srcp.reshape(NW, NCHUNK, CB)
    dst3d = dstp.reshape(NW, NCHUNK, CB)
    z128 = jnp.zeros((NPAD, D), jnp.float32)
    ones_h = jnp.ones((CB, D), jnp.float32)

    (deg,) = _sc_deg(dst3d, z128, ones_h)
    (agg1,) = _sc_agg(x, src3d, dst3d, z128)
    h = _tc_layer(x, agg1, deg, Wl1, Wr1, b1, relu=True)
    (agg2,) = _sc_agg(h, src3d, dst3d, z128)
    out = _tc_layer(h, agg2, deg, Wl2, Wr2, b2, relu=False)
    return out
